# baseline (speedup 1.0000x reference)
import functools

import jax
import jax.numpy as jnp
from jax import lax
from jax.experimental import pallas as pl
from jax.experimental.pallas import tpu as pltpu
from jax.experimental.pallas import tpu_sc as plsc

_NUM_CORES = 2
_NUM_SUBCORES = 16
_NW = _NUM_CORES * _NUM_SUBCORES
_CHUNK = 128


@functools.partial(jax.jit, static_argnums=(2, 3))
def _gather_sc(weight, idx3, b_per_w, d):
    n_chunks = b_per_w // _CHUNK
    mesh = plsc.VectorSubcoreMesh(core_axis_name="c", subcore_axis_name="s")

    @functools.partial(
        pl.kernel,
        mesh=mesh,
        compiler_params=pltpu.CompilerParams(use_tc_tiling_on_sc=False),
        out_type=jax.ShapeDtypeStruct((_NW * b_per_w, d), jnp.float32),
        scratch_types=[
            pltpu.VMEM((n_chunks, _CHUNK), jnp.int32),
            pltpu.VMEM((b_per_w, d), jnp.float32),
            pltpu.SemaphoreType.DMA,
        ],
    )
    def k(table_hbm, idx_hbm, out_hbm, idx_v, rows_v, sem):
        wid = lax.axis_index("s") * _NUM_CORES + lax.axis_index("c")
        base = wid * b_per_w
        pltpu.sync_copy(idx_hbm.at[wid], idx_v)
        copies = [
            pltpu.async_copy(
                table_hbm.at[idx_v.at[j]],
                rows_v.at[pl.ds(j * _CHUNK, _CHUNK)],
                sem,
            )
            for j in range(n_chunks)
        ]
        for c in copies:
            c.wait()
        pltpu.sync_copy(rows_v, out_hbm.at[pl.ds(base, b_per_w)])

    return k(weight, idx3)


def kernel(x, weight):
    b = x.shape[0]
    d = weight.shape[1]
    b_per_w = b // _NW
    idx3 = x.reshape(_NW, b_per_w // _CHUNK, _CHUNK)
    return _gather_sc(weight, idx3, b_per_w, d)


# speedup vs baseline: 2.5016x; 2.5016x over previous
import functools

import jax
import jax.numpy as jnp
from jax import lax
from jax.experimental import pallas as pl
from jax.experimental.pallas import tpu as pltpu
from jax.experimental.pallas import tpu_sc as plsc

_NUM_CORES = 2
_NUM_SUBCORES = 16
_NW = _NUM_CORES * _NUM_SUBCORES
_CHUNK = 64
_L = 16


@functools.partial(jax.jit, static_argnums=(2, 3))
def _gather_sc(weight3, idx3, b_per_w, d):
    n_chunks = b_per_w // _CHUNK
    mesh = plsc.VectorSubcoreMesh(core_axis_name="c", subcore_axis_name="s")

    @functools.partial(
        pl.kernel,
        mesh=mesh,
        out_type=jax.ShapeDtypeStruct((_NW * b_per_w, d), jnp.float32),
        scratch_types=[
            pltpu.VMEM((n_chunks, _CHUNK), jnp.int32),
            pltpu.VMEM((_CHUNK, d), jnp.float32),
            pltpu.SemaphoreType.DMA,
        ],
    )
    def k(table_hbm, idx_hbm, out_hbm, idx_v, rowchunk_v, sem):
        wid = lax.axis_index("s") * _NUM_CORES + lax.axis_index("c")
        base = wid * b_per_w
        pltpu.sync_copy(idx_hbm.at[wid], idx_v)

        def chunk_body(j, carry):
            copies = []
            for g in range(_CHUNK // _L):
                xv = idx_v[j, pl.ds(g * _L, _L)]
                for l in range(_L):
                    x = xv[l]
                    q = lax.shift_right_logical(x, 3)
                    r = lax.rem(x, 8)
                    copies.append(
                        pltpu.async_copy(
                            table_hbm.at[q, r],
                            rowchunk_v.at[g * _L + l],
                            sem,
                        )
                    )
            for c in copies:
                c.wait()
            pltpu.sync_copy(
                rowchunk_v, out_hbm.at[pl.ds(base + j * _CHUNK, _CHUNK)]
            )
            return carry

        lax.fori_loop(0, n_chunks, chunk_body, 0)

    return k(weight3, idx3)


def kernel(x, weight):
    b = x.shape[0]
    d = weight.shape[1]
    b_per_w = b // _NW
    weight3 = weight.reshape(-1, 8, d)
    idx3 = x.reshape(_NW, b_per_w // _CHUNK, _CHUNK)
    return _gather_sc(weight3, idx3, b_per_w, d)
